# Initial kernel scaffold; baseline (speedup 1.0000x reference)
import jax
import jax.numpy as jnp
from jax.experimental import pallas as pl


def kernel(x_atom, bond_idx, angle_idx, torsion_idx, bond_repr, angle_repr, torsion_repr, bond_W1, bond_b1, bond_W2, bond_b2, bond_W3, bond_b3, angle_W1, angle_b1, angle_W2, angle_b2, angle_W3, angle_b3, torsion_W1, torsion_b1, torsion_W2, torsion_b2, torsion_W3, torsion_b3):
    raise NotImplementedError("write your pallas kernel here")


# speedup vs baseline: 2.4746x; 2.4746x over previous
import functools

import jax
import jax.numpy as jnp
from jax import lax
from jax.experimental import pallas as pl
from jax.experimental.pallas import tpu as pltpu
from jax.experimental.pallas import tpu_sc as plsc

_H = 64
_D = 128
_NOUT = 10
_K = 112
_NW = 32
_ROWS = 1000


def _proj_body(x_ref, wb_ref, wa13_ref, wa2_ref, wt01_ref, wt23_ref,
               tb_ref, ta13_ref, ta2_ref, tt01_ref, tt23_ref):
    x = x_ref[...]
    tb_ref[...] = jnp.dot(x, wb_ref[...], preferred_element_type=jnp.float32)
    ta13_ref[...] = jnp.dot(x, wa13_ref[...], preferred_element_type=jnp.float32)
    ta2_ref[...] = jnp.dot(x, wa2_ref[...], preferred_element_type=jnp.float32)
    tt01_ref[...] = jnp.dot(x, wt01_ref[...], preferred_element_type=jnp.float32)
    tt23_ref[...] = jnp.dot(x, wt23_ref[...], preferred_element_type=jnp.float32)


def _project(x_atom, w_bond, w_a13, w_a2, w_t01, w_t23):
    n = x_atom.shape[0]
    grid = n // _ROWS
    rowspec = pl.BlockSpec((_ROWS, _D), lambda i: (i, 0))
    wspec = pl.BlockSpec((_D, _D), lambda i: (0, 0))
    return pl.pallas_call(
        _proj_body,
        grid=(grid,),
        in_specs=[rowspec] + [wspec] * 5,
        out_specs=[rowspec] * 5,
        out_shape=[jax.ShapeDtypeStruct((n, _D), jnp.float32)] * 5,
    )(x_atom, w_bond, w_a13, w_a2, w_t01, w_t23)


def _sc_mesh():
    return plsc.VectorSubcoreMesh(core_axis_name="c", subcore_axis_name="s")


def _bond_gather(npad):
    c_per_w = npad // _NW
    nchunk = c_per_w // _K
    nc = plsc.get_sparse_core_info().num_cores

    @functools.partial(
        pl.kernel, mesh=_sc_mesh(),
        out_type=jax.ShapeDtypeStruct((npad, _D), jnp.float32),
        scratch_types=[
            pltpu.VMEM((_K,), jnp.int32), pltpu.VMEM((_K,), jnp.int32),
            pltpu.VMEM((_K, _D), jnp.float32),
            pltpu.VMEM((_K, _D), jnp.float32),
            pltpu.VMEM((_K, _D), jnp.float32),
            pltpu.SemaphoreType.DMA, pltpu.SemaphoreType.DMA,
        ])
    def k(tb, i0h, i1h, out_h, i0v, i1v, r0, r1, ov, s0, s1):
        wid = lax.axis_index("s") * nc + lax.axis_index("c")
        base = wid * c_per_w

        def chunk(cidx, carry):
            off = base + cidx * _K
            pltpu.sync_copy(i0h.at[pl.ds(off, _K)], i0v)
            pltpu.sync_copy(i1h.at[pl.ds(off, _K)], i1v)
            cp0 = pltpu.async_copy(tb.at[i0v], r0, s0)
            cp1 = pltpu.async_copy(tb.at[i1v], r1, s1)
            cp0.wait()
            cp1.wait()

            def row(i, carry2):
                for j in range(4):
                    lo = pl.ds(j * 16, 16)
                    hi = pl.ds(_H + j * 16, 16)
                    ov[i, lo] = r0[i, lo] + r1[i, hi]
                    ov[i, hi] = r1[i, lo] + r0[i, hi]
                return carry2

            lax.fori_loop(0, _K, row, 0)
            pltpu.sync_copy(ov, out_h.at[pl.ds(off, _K)])
            return carry

        lax.fori_loop(0, nchunk, chunk, 0)

    return k


def _angle_gather(npad):
    c_per_w = npad // _NW
    nchunk = c_per_w // _K
    nc = plsc.get_sparse_core_info().num_cores

    @functools.partial(
        pl.kernel, mesh=_sc_mesh(),
        out_type=jax.ShapeDtypeStruct((npad, _D), jnp.float32),
        scratch_types=[
            pltpu.VMEM((_K,), jnp.int32), pltpu.VMEM((_K,), jnp.int32),
            pltpu.VMEM((_K,), jnp.int32),
            pltpu.VMEM((_K, _D), jnp.float32),
            pltpu.VMEM((_K, _D), jnp.float32),
            pltpu.VMEM((_K, _D), jnp.float32),
            pltpu.VMEM((_K, _D), jnp.float32),
            pltpu.SemaphoreType.DMA, pltpu.SemaphoreType.DMA,
            pltpu.SemaphoreType.DMA,
        ])
    def k(t13, t2, i0h, i1h, i2h, out_h, i0v, i1v, i2v, u0, u2, a2v, ov,
          s0, s1, s2):
        wid = lax.axis_index("s") * nc + lax.axis_index("c")
        base = wid * c_per_w

        def chunk(cidx, carry):
            off = base + cidx * _K
            pltpu.sync_copy(i0h.at[pl.ds(off, _K)], i0v)
            pltpu.sync_copy(i1h.at[pl.ds(off, _K)], i1v)
            pltpu.sync_copy(i2h.at[pl.ds(off, _K)], i2v)
            cp0 = pltpu.async_copy(t13.at[i0v], u0, s0)
            cp1 = pltpu.async_copy(t2.at[i1v], a2v, s1)
            cp2 = pltpu.async_copy(t13.at[i2v], u2, s2)
            cp0.wait()
            cp1.wait()
            cp2.wait()

            def row(i, carry2):
                for j in range(4):
                    lo = pl.ds(j * 16, 16)
                    hi = pl.ds(_H + j * 16, 16)
                    mid = a2v[i, lo]
                    ov[i, lo] = u0[i, lo] + mid + u2[i, hi]
                    ov[i, hi] = u2[i, lo] + mid + u0[i, hi]
                return carry2

            lax.fori_loop(0, _K, row, 0)
            pltpu.sync_copy(ov, out_h.at[pl.ds(off, _K)])
            return carry

        lax.fori_loop(0, nchunk, chunk, 0)

    return k


def _torsion_gather(npad):
    c_per_w = npad // _NW
    nchunk = c_per_w // _K
    nc = plsc.get_sparse_core_info().num_cores

    @functools.partial(
        pl.kernel, mesh=_sc_mesh(),
        out_type=jax.ShapeDtypeStruct((npad, _D), jnp.float32),
        scratch_types=[
            pltpu.VMEM((_K,), jnp.int32), pltpu.VMEM((_K,), jnp.int32),
            pltpu.VMEM((_K,), jnp.int32), pltpu.VMEM((_K,), jnp.int32),
            pltpu.VMEM((_K, _D), jnp.float32),
            pltpu.VMEM((_K, _D), jnp.float32),
            pltpu.VMEM((_K, _D), jnp.float32),
            pltpu.VMEM((_K, _D), jnp.float32),
            pltpu.VMEM((_K, _D), jnp.float32),
            pltpu.SemaphoreType.DMA, pltpu.SemaphoreType.DMA,
            pltpu.SemaphoreType.DMA, pltpu.SemaphoreType.DMA,
        ])
    def k(t01, t23, i0h, i1h, i2h, i3h, out_h, i0v, i1v, i2v, i3v,
          r0, r1, r2, r3, ov, s0, s1, s2, s3):
        wid = lax.axis_index("s") * nc + lax.axis_index("c")
        base = wid * c_per_w

        def chunk(cidx, carry):
            off = base + cidx * _K
            pltpu.sync_copy(i0h.at[pl.ds(off, _K)], i0v)
            pltpu.sync_copy(i1h.at[pl.ds(off, _K)], i1v)
            pltpu.sync_copy(i2h.at[pl.ds(off, _K)], i2v)
            pltpu.sync_copy(i3h.at[pl.ds(off, _K)], i3v)
            cp0 = pltpu.async_copy(t01.at[i0v], r0, s0)
            cp1 = pltpu.async_copy(t01.at[i1v], r1, s1)
            cp2 = pltpu.async_copy(t23.at[i2v], r2, s2)
            cp3 = pltpu.async_copy(t23.at[i3v], r3, s3)
            cp0.wait()
            cp1.wait()
            cp2.wait()
            cp3.wait()

            def row(i, carry2):
                for j in range(4):
                    lo = pl.ds(j * 16, 16)
                    hi = pl.ds(_H + j * 16, 16)
                    g = (r0[i, lo] + r1[i, hi]) + (r2[i, lo] + r3[i, hi])
                    ov[i, lo] = g
                    ov[i, hi] = g
                return carry2

            lax.fori_loop(0, _K, row, 0)
            pltpu.sync_copy(ov, out_h.at[pl.ds(off, _K)])
            return carry

        lax.fori_loop(0, nchunk, chunk, 0)

    return k


def _mlp_body(bg, ag, tg, br, ar, tr,
              bw1, bb1, bw2, bb2, bw3, bb3,
              aw1, ab1, aw2, ab2, aw3, ab3,
              tw1, tb1, tw2, tb2, tw3, tb3,
              bo, ao, to):
    def tail(g, rep, w1, b1, w2, b2, w3, b3):
        h = jax.nn.relu(g + rep * w1 + b1)
        h = jax.nn.relu(jnp.dot(h, w2, preferred_element_type=jnp.float32) + b2)
        return jnp.dot(h, w3, preferred_element_type=jnp.float32) + b3

    bo[...] = tail(bg[...], br[...], bw1[...], bb1[...], bw2[...], bb2[...],
                   bw3[...], bb3[...])
    ao[...] = tail(ag[...], ar[...], aw1[...], ab1[...], aw2[...], ab2[...],
                   aw3[...], ab3[...])
    to[...] = tail(tg[...], tr[...], tw1[...], tb1[...], tw2[...], tb2[...],
                   tw3[...], tb3[...])


def _mlp(n, bg, ag, tg, br, ar, tr, *weights):
    grid = n // _ROWS
    gspec = pl.BlockSpec((_ROWS, _D), lambda i: (i, 0))
    rspec = pl.BlockSpec((_ROWS, 1), lambda i: (i, 0))
    w1spec = pl.BlockSpec((1, _D), lambda i: (0, 0))
    w2spec = pl.BlockSpec((_D, _D), lambda i: (0, 0))
    w3spec = pl.BlockSpec((_D, _NOUT), lambda i: (0, 0))
    b3spec = pl.BlockSpec((1, _NOUT), lambda i: (0, 0))
    ospec = pl.BlockSpec((_ROWS, _NOUT), lambda i: (i, 0))
    tspec = [w1spec, w1spec, w2spec, w1spec, w3spec, b3spec]
    return pl.pallas_call(
        _mlp_body,
        grid=(grid,),
        in_specs=[gspec, gspec, gspec, rspec, rspec, rspec] + tspec * 3,
        out_specs=[ospec, ospec, ospec],
        out_shape=[jax.ShapeDtypeStruct((n, _NOUT), jnp.float32)] * 3,
    )(bg, ag, tg, br, ar, tr, *weights)


def kernel(x_atom, bond_idx, angle_idx, torsion_idx, bond_repr, angle_repr,
           torsion_repr, bond_W1, bond_b1, bond_W2, bond_b2, bond_W3, bond_b3,
           angle_W1, angle_b1, angle_W2, angle_b2, angle_W3, angle_b3,
           torsion_W1, torsion_b1, torsion_W2, torsion_b2, torsion_W3,
           torsion_b3):
    n = bond_idx.shape[0]
    span = _NW * _K
    npad = -(-n // span) * span

    w_bond = jnp.concatenate([bond_W1[:_D], bond_W1[_D:2 * _D]], axis=1)
    w_a13 = jnp.concatenate([angle_W1[:_D], angle_W1[2 * _D:3 * _D]], axis=1)
    w_a2 = jnp.concatenate([angle_W1[_D:2 * _D]] * 2, axis=1)
    w_t01 = jnp.concatenate([torsion_W1[:_D], torsion_W1[_D:2 * _D]], axis=1)
    w_t23 = jnp.concatenate([torsion_W1[2 * _D:3 * _D],
                             torsion_W1[3 * _D:4 * _D]], axis=1)

    tb, ta13, ta2, tt01, tt23 = _project(x_atom, w_bond, w_a13, w_a2,
                                         w_t01, w_t23)

    pad = npad - n

    def prep(idx, col):
        return jnp.pad(idx[:, col].astype(jnp.int32), (0, pad))

    b0, b1i = prep(bond_idx, 0), prep(bond_idx, 1)
    a0, a1i, a2i = (prep(angle_idx, c) for c in range(3))
    t0, t1i, t2i, t3i = (prep(torsion_idx, c) for c in range(4))

    bg = _bond_gather(npad)(tb, b0, b1i)
    ag = _angle_gather(npad)(ta13, ta2, a0, a1i, a2i)
    tg = _torsion_gather(npad)(tt01, tt23, t0, t1i, t2i, t3i)

    zeros_h = jnp.zeros((_H, _H), jnp.float32)

    def dup1(v):
        return jnp.concatenate([v.reshape(1, -1)] * 2, axis=1)

    def blkdiag(w2a, w2b):
        return jnp.concatenate(
            [jnp.concatenate([w2a, zeros_h], axis=1),
             jnp.concatenate([zeros_h, w2b], axis=1)], axis=0)

    wtail = (
        dup1(bond_W1[2 * _D]), dup1(bond_b1),
        blkdiag(bond_W2, bond_W2), dup1(bond_b2),
        jnp.concatenate([bond_W3, bond_W3], axis=0),
        (2.0 * bond_b3).reshape(1, _NOUT),
        dup1(angle_W1[3 * _D]), dup1(angle_b1),
        blkdiag(angle_W2, angle_W2), dup1(angle_b2),
        jnp.concatenate([angle_W3, angle_W3], axis=0),
        (2.0 * angle_b3).reshape(1, _NOUT),
        dup1(torsion_W1[4 * _D]), dup1(torsion_b1),
        blkdiag(torsion_W2, zeros_h),
        jnp.concatenate([torsion_b2.reshape(1, _H),
                         jnp.zeros((1, _H), jnp.float32)], axis=1),
        jnp.concatenate([torsion_W3, jnp.zeros((_H, _NOUT), jnp.float32)],
                        axis=0),
        torsion_b3.reshape(1, _NOUT),
    )

    bo, ao, to = _mlp(n, bg, ag, tg, bond_repr, angle_repr, torsion_repr,
                      *wtail)
    return (bo, ao, to)


# speedup vs baseline: 2.6072x; 1.0536x over previous
import functools

import jax
import jax.numpy as jnp
from jax import lax
from jax.experimental import pallas as pl
from jax.experimental.pallas import tpu as pltpu
from jax.experimental.pallas import tpu_sc as plsc

_H = 64
_D = 128
_NOUT = 10
_K = 56
_NW = 32
_ROWS = 1000


def _proj_body(x_ref, wb_ref, wa13_ref, wa2_ref, wt01_ref, wt23_ref,
               tb_ref, ta13_ref, ta2_ref, tt01_ref, tt23_ref):
    x = x_ref[...]
    tb_ref[...] = jnp.dot(x, wb_ref[...], preferred_element_type=jnp.float32)
    ta13_ref[...] = jnp.dot(x, wa13_ref[...], preferred_element_type=jnp.float32)
    ta2_ref[...] = jnp.dot(x, wa2_ref[...], preferred_element_type=jnp.float32)
    tt01_ref[...] = jnp.dot(x, wt01_ref[...], preferred_element_type=jnp.float32)
    tt23_ref[...] = jnp.dot(x, wt23_ref[...], preferred_element_type=jnp.float32)


def _project(x_atom, w_bond, w_a13, w_a2, w_t01, w_t23):
    n = x_atom.shape[0]
    grid = n // _ROWS
    rowspec = pl.BlockSpec((_ROWS, _D), lambda i: (i, 0))
    wspec = pl.BlockSpec((_D, _D), lambda i: (0, 0))
    return pl.pallas_call(
        _proj_body,
        grid=(grid,),
        in_specs=[rowspec] + [wspec] * 5,
        out_specs=[rowspec] * 5,
        out_shape=[jax.ShapeDtypeStruct((n, _D), jnp.float32)] * 5,
    )(x_atom, w_bond, w_a13, w_a2, w_t01, w_t23)


def _sc_mesh():
    return plsc.VectorSubcoreMesh(core_axis_name="c", subcore_axis_name="s")


def _fused_gather(npad):
    c_per_w = npad // _NW
    nchunk = c_per_w // _K
    half = nchunk // 2
    nc = plsc.get_sparse_core_info().num_cores

    scratch = ([pltpu.VMEM((_K,), jnp.int32)] * 8
               + [pltpu.VMEM((_K, _D), jnp.float32)] * 10
               + [pltpu.SemaphoreType.DMA] * 4)

    @functools.partial(
        pl.kernel, mesh=_sc_mesh(),
        out_type=(jax.ShapeDtypeStruct((npad, _D), jnp.float32),
                  jax.ShapeDtypeStruct((npad, _D), jnp.float32),
                  jax.ShapeDtypeStruct((npad, _D), jnp.float32)),
        scratch_types=scratch)
    def k(tb, ta13, ta2, tt01, tt23,
          b0h, b1h, a0h, a1h, a2h, t0h, t1h, t2h, t3h,
          bg_h, ag_h, tg_h, *scr):
        i_v = (scr[0:4], scr[4:8])
        r_v = (scr[8:12], scr[12:16])
        ov_v = scr[16:18]
        gsem = scr[18:20]
        osem = scr[20:22]
        wid = lax.axis_index("s") * nc + lax.axis_index("c")
        base = wid * c_per_w

        def run_phase(tables, idx_hs, out_h, valu_row, first_phase):
            nidx = len(tables)

            def issue(c, slot):
                off = base + c * _K
                for q in range(nidx):
                    pltpu.sync_copy(idx_hs[q].at[pl.ds(off, _K)],
                                    i_v[slot][q])
                for q in range(nidx):
                    pltpu.async_copy(tables[q].at[i_v[slot][q]],
                                     r_v[slot][q], gsem[slot])

            def wait_gathers(slot):
                for q in range(nidx):
                    pltpu.make_async_copy(tables[q].at[i_v[slot][q]],
                                          r_v[slot][q], gsem[slot]).wait()

            def wait_out(slot, off):
                pltpu.make_async_copy(
                    ov_v[slot], out_h.at[pl.ds(off, _K)], osem[slot]).wait()

            issue(0, 0)
            issue(1, 1)

            def body(g, carry):
                for slot in (0, 1):
                    c = 2 * g + slot
                    off = base + c * _K
                    wait_gathers(slot)
                    if first_phase:
                        @pl.when(g > 0)
                        def _():
                            wait_out(slot, off)
                    else:
                        wait_out(slot, off)

                    def row(i, carry2):
                        valu_row(i, slot)
                        return carry2

                    lax.fori_loop(0, _K, row, 0)
                    pltpu.async_copy(ov_v[slot], out_h.at[pl.ds(off, _K)],
                                     osem[slot])

                    @pl.when(g < half - 1)
                    def _():
                        issue(c + 2, slot)
                return carry

            lax.fori_loop(0, half, body, 0)

        def bond_row(i, slot):
            r0, r1 = r_v[slot][0], r_v[slot][1]
            ov = ov_v[slot]
            for j in range(4):
                lo = pl.ds(j * 16, 16)
                hi = pl.ds(_H + j * 16, 16)
                ov[i, lo] = r0[i, lo] + r1[i, hi]
                ov[i, hi] = r1[i, lo] + r0[i, hi]

        def angle_row(i, slot):
            u0, a2v, u2 = r_v[slot][0], r_v[slot][1], r_v[slot][2]
            ov = ov_v[slot]
            for j in range(4):
                lo = pl.ds(j * 16, 16)
                hi = pl.ds(_H + j * 16, 16)
                mid = a2v[i, lo]
                ov[i, lo] = u0[i, lo] + mid + u2[i, hi]
                ov[i, hi] = u2[i, lo] + mid + u0[i, hi]

        def torsion_row(i, slot):
            r0, r1, r2, r3 = r_v[slot]
            ov = ov_v[slot]
            for j in range(4):
                lo = pl.ds(j * 16, 16)
                hi = pl.ds(_H + j * 16, 16)
                g = (r0[i, lo] + r1[i, hi]) + (r2[i, lo] + r3[i, hi])
                ov[i, lo] = g
                ov[i, hi] = g

        run_phase((tb, tb), (b0h, b1h), bg_h, bond_row, True)
        run_phase((ta13, ta2, ta13), (a0h, a1h, a2h), ag_h, angle_row, False)
        run_phase((tt01, tt01, tt23, tt23), (t0h, t1h, t2h, t3h), tg_h,
                  torsion_row, False)

        pltpu.make_async_copy(ov_v[0], tg_h.at[pl.ds(base, _K)],
                              osem[0]).wait()
        pltpu.make_async_copy(ov_v[1], tg_h.at[pl.ds(base, _K)],
                              osem[1]).wait()

    return k


def _mlp_body(bg, ag, tg, br, ar, tr,
              bw1, bb1, bw2, bb2, bw3, bb3,
              aw1, ab1, aw2, ab2, aw3, ab3,
              tw1, tb1, tw2, tb2, tw3, tb3,
              bo, ao, to):
    def tail(g, rep, w1, b1, w2, b2, w3, b3):
        h = jax.nn.relu(g + rep * w1 + b1)
        h = jax.nn.relu(jnp.dot(h, w2, preferred_element_type=jnp.float32) + b2)
        return jnp.dot(h, w3, preferred_element_type=jnp.float32) + b3

    bo[...] = tail(bg[...], br[...], bw1[...], bb1[...], bw2[...], bb2[...],
                   bw3[...], bb3[...])
    ao[...] = tail(ag[...], ar[...], aw1[...], ab1[...], aw2[...], ab2[...],
                   aw3[...], ab3[...])
    to[...] = tail(tg[...], tr[...], tw1[...], tb1[...], tw2[...], tb2[...],
                   tw3[...], tb3[...])


def _mlp(n, bg, ag, tg, br, ar, tr, *weights):
    grid = n // _ROWS
    gspec = pl.BlockSpec((_ROWS, _D), lambda i: (i, 0))
    rspec = pl.BlockSpec((_ROWS, 1), lambda i: (i, 0))
    w1spec = pl.BlockSpec((1, _D), lambda i: (0, 0))
    w2spec = pl.BlockSpec((_D, _D), lambda i: (0, 0))
    w3spec = pl.BlockSpec((_D, _NOUT), lambda i: (0, 0))
    b3spec = pl.BlockSpec((1, _NOUT), lambda i: (0, 0))
    ospec = pl.BlockSpec((_ROWS, _NOUT), lambda i: (i, 0))
    tspec = [w1spec, w1spec, w2spec, w1spec, w3spec, b3spec]
    return pl.pallas_call(
        _mlp_body,
        grid=(grid,),
        in_specs=[gspec, gspec, gspec, rspec, rspec, rspec] + tspec * 3,
        out_specs=[ospec, ospec, ospec],
        out_shape=[jax.ShapeDtypeStruct((n, _NOUT), jnp.float32)] * 3,
    )(bg, ag, tg, br, ar, tr, *weights)


def kernel(x_atom, bond_idx, angle_idx, torsion_idx, bond_repr, angle_repr,
           torsion_repr, bond_W1, bond_b1, bond_W2, bond_b2, bond_W3, bond_b3,
           angle_W1, angle_b1, angle_W2, angle_b2, angle_W3, angle_b3,
           torsion_W1, torsion_b1, torsion_W2, torsion_b2, torsion_W3,
           torsion_b3):
    n = bond_idx.shape[0]
    span = _NW * _K
    npad = -(-n // span) * span

    w_bond = jnp.concatenate([bond_W1[:_D], bond_W1[_D:2 * _D]], axis=1)
    w_a13 = jnp.concatenate([angle_W1[:_D], angle_W1[2 * _D:3 * _D]], axis=1)
    w_a2 = jnp.concatenate([angle_W1[_D:2 * _D]] * 2, axis=1)
    w_t01 = jnp.concatenate([torsion_W1[:_D], torsion_W1[_D:2 * _D]], axis=1)
    w_t23 = jnp.concatenate([torsion_W1[2 * _D:3 * _D],
                             torsion_W1[3 * _D:4 * _D]], axis=1)

    tb, ta13, ta2, tt01, tt23 = _project(x_atom, w_bond, w_a13, w_a2,
                                         w_t01, w_t23)

    pad = npad - n

    def prep(idx, col):
        return jnp.pad(idx[:, col].astype(jnp.int32), (0, pad))

    b0, b1i = prep(bond_idx, 0), prep(bond_idx, 1)
    a0, a1i, a2i = (prep(angle_idx, c) for c in range(3))
    t0, t1i, t2i, t3i = (prep(torsion_idx, c) for c in range(4))

    bg, ag, tg = _fused_gather(npad)(tb, ta13, ta2, tt01, tt23,
                                     b0, b1i, a0, a1i, a2i,
                                     t0, t1i, t2i, t3i)

    zeros_h = jnp.zeros((_H, _H), jnp.float32)

    def dup1(v):
        return jnp.concatenate([v.reshape(1, -1)] * 2, axis=1)

    def blkdiag(w2a, w2b):
        return jnp.concatenate(
            [jnp.concatenate([w2a, zeros_h], axis=1),
             jnp.concatenate([zeros_h, w2b], axis=1)], axis=0)

    wtail = (
        dup1(bond_W1[2 * _D]), dup1(bond_b1),
        blkdiag(bond_W2, bond_W2), dup1(bond_b2),
        jnp.concatenate([bond_W3, bond_W3], axis=0),
        (2.0 * bond_b3).reshape(1, _NOUT),
        dup1(angle_W1[3 * _D]), dup1(angle_b1),
        blkdiag(angle_W2, angle_W2), dup1(angle_b2),
        jnp.concatenate([angle_W3, angle_W3], axis=0),
        (2.0 * angle_b3).reshape(1, _NOUT),
        dup1(torsion_W1[4 * _D]), dup1(torsion_b1),
        blkdiag(torsion_W2, zeros_h),
        jnp.concatenate([torsion_b2.reshape(1, _H),
                         jnp.zeros((1, _H), jnp.float32)], axis=1),
        jnp.concatenate([torsion_W3, jnp.zeros((_H, _NOUT), jnp.float32)],
                        axis=0),
        torsion_b3.reshape(1, _NOUT),
    )

    bo, ao, to = _mlp(n, bg, ag, tg, bond_repr, angle_repr, torsion_repr,
                      *wtail)
    return (bo, ao, to)
